# baseline (speedup 1.0000x reference)
import functools
import jax
import jax.numpy as jnp
from jax import lax
from jax.experimental import pallas as pl
from jax.experimental.pallas import tpu as pltpu
from jax.experimental.pallas import tpu_sc as plsc

NC = 2
NS = 16
NW = NC * NS
LN = 16

F32 = jnp.float32


def _sc_mesh():
    return plsc.VectorSubcoreMesh(core_axis_name="c", subcore_axis_name="s",
                                  num_cores=NC, num_subcores=NS)


CH = 96


def _make_deg_kernel(npad, n_chunk_rows, b):
    rows_per_tile = n_chunk_rows // NW
    zrows = npad // NS
    bps = b // NS

    @functools.partial(
        pl.kernel,
        out_type=(
            jax.ShapeDtypeStruct((NC * npad,), F32),
            jax.ShapeDtypeStruct((npad,), F32),
        ),
        mesh=_sc_mesh(),
        scratch_types=[
            pltpu.VMEM((rows_per_tile, 1, CH), jnp.int32),
            pltpu.VMEM((CH,), F32),
            pltpu.VMEM((zrows,), F32),
            pltpu.VMEM((bps,), jnp.int32),
            pltpu.VMEM_SHARED((npad,), F32),
            pltpu.VMEM_SHARED((npad,), F32),
        ],
    )
    def deg_kernel(dst3, tgt, out, flags_o, idx_v, ones_v, zbuf, tgt_v,
                   cnt_sh, flag_sh):
        c = lax.axis_index("c")
        s = lax.axis_index("s")
        wid = s * NC + c

        def zb(i, carry):
            zbuf[pl.ds(i * LN, LN)] = jnp.zeros((LN,), F32)
            return carry
        lax.fori_loop(0, zrows // LN, zb, 0)
        for k in range(CH // LN):
            ones_v[pl.ds(k * LN, LN)] = jnp.ones((LN,), F32)
        pltpu.sync_copy(zbuf, cnt_sh.at[pl.ds(s * zrows, zrows)])
        pltpu.sync_copy(zbuf, flag_sh.at[pl.ds(s * zrows, zrows)])
        plsc.subcore_barrier()

        pltpu.sync_copy(tgt.at[pl.ds(s * bps, bps)], tgt_v)
        pltpu.sync_copy(ones_v.at[pl.ds(0, bps)], flag_sh.at[tgt_v], add=True)

        pltpu.sync_copy(dst3.at[pl.ds(wid * rows_per_tile, rows_per_tile)], idx_v)

        def body(j, carry):
            pltpu.sync_copy(ones_v, cnt_sh.at[idx_v.at[j, 0]], add=True)
            return carry
        lax.fori_loop(0, rows_per_tile, body, 0)

        plsc.subcore_barrier()
        pltpu.sync_copy(cnt_sh.at[pl.ds(s * zrows, zrows)],
                        out.at[pl.ds(c * npad + s * zrows, zrows)])

        @pl.when(c == 0)
        def _():
            pltpu.sync_copy(flag_sh.at[pl.ds(s * zrows, zrows)],
                            flags_o.at[pl.ds(s * zrows, zrows)])

    return deg_kernel


def _make_agg_kernel(npad, n_chunk_rows):
    rows_per_tile = n_chunk_rows // NS
    zrows = npad // NS

    @functools.partial(
        pl.kernel,
        out_type=jax.ShapeDtypeStruct((NC * npad, 128), F32),
        mesh=_sc_mesh(),
        scratch_types=[
            pltpu.VMEM((2, 1, CH), jnp.int32),
            pltpu.VMEM((2, 1, CH), jnp.int32),
            pltpu.VMEM((2, CH, 128), F32),
            pltpu.VMEM_SHARED((npad, 128), F32),
            pltpu.SemaphoreType.DMA,
            pltpu.SemaphoreType.DMA,
        ],
    )
    def agg_kernel(table, src3, dst3, out, sidx, didx, rows, acc_sh,
                   gsem, isem):
        c = lax.axis_index("c")
        s = lax.axis_index("s")

        def zb(i, carry):
            for k in range(128 // LN):
                rows[0, i, pl.ds(k * LN, LN)] = jnp.zeros((LN,), F32)
            return carry
        lax.fori_loop(0, CH, zb, 0)
        for r in range(zrows // CH):
            pltpu.sync_copy(rows.at[0], acc_sh.at[pl.ds(s * zrows + r * CH, CH)])
        plsc.subcore_barrier()

        base = s * rows_per_tile
        off = c * npad

        def fetch(j, p):
            pltpu.make_async_copy(src3.at[pl.ds(base + j, 1)],
                                  sidx.at[pl.ds(p, 1)], isem).start()
            pltpu.make_async_copy(dst3.at[pl.ds(base + j, 1)],
                                  didx.at[pl.ds(p, 1)], isem).start()

        def drain_idx():
            pltpu.make_async_copy(src3.at[pl.ds(0, 1)],
                                  sidx.at[pl.ds(0, 1)], isem).wait()
            pltpu.make_async_copy(dst3.at[pl.ds(0, 1)],
                                  didx.at[pl.ds(0, 1)], isem).wait()

        def shift(p):
            for k in range(CH // LN):
                sl = pl.ds(k * LN, LN)
                sidx[p, 0, sl] = sidx[p, 0, sl] + off

        def start_gather(p):
            pltpu.make_async_copy(table.at[sidx.at[p, 0]], rows.at[p],
                                  gsem).start()

        def wait_gather(p):
            pltpu.make_async_copy(table.at[pl.ds(0, CH)], rows.at[p],
                                  gsem).wait()

        fetch(0, 0)
        drain_idx()
        shift(0)
        start_gather(0)
        fetch(1, 1)

        def body(jj, carry):
            for p in range(2):
                j = jj * 2 + p
                wait_gather(p)

                @pl.when(j + 1 < rows_per_tile)
                def _():
                    drain_idx()
                    shift(1 - p)
                    start_gather(1 - p)
                pltpu.sync_copy(rows.at[p], acc_sh.at[didx.at[p, 0]], add=True)

                @pl.when(j + 2 < rows_per_tile)
                def _():
                    fetch(j + 2, p)
            return carry
        lax.fori_loop(0, rows_per_tile // 2, body, 0)

        plsc.subcore_barrier()
        for r in range(zrows // 128):
            pltpu.sync_copy(acc_sh.at[pl.ds(s * zrows + r * 128, 128)],
                            out.at[pl.ds(c * npad + s * zrows + r * 128, 128)])

    return agg_kernel


def _make_filter_kernel(npad, n_chunk_rows, b, dump):
    cpt = n_chunk_rows // NW
    cap = cpt * CH
    I32 = jnp.int32

    rcap = cap + CH

    @functools.partial(
        pl.kernel,
        out_type=(
            jax.ShapeDtypeStruct((NW * rcap,), I32),
            jax.ShapeDtypeStruct((NW * rcap,), I32),
            jax.ShapeDtypeStruct((NW * LN,), I32),
        ),
        mesh=_sc_mesh(),
        scratch_types=[
            pltpu.VMEM((cpt, 1, CH), I32),
            pltpu.VMEM((cpt, 1, CH), I32),
            pltpu.VMEM((cpt, 1, CH), I32),
            pltpu.VMEM((2, 1, CH), F32),
            pltpu.VMEM((cap,), I32),
            pltpu.VMEM((LN,), I32),
            pltpu.VMEM((2 * LN,), I32),
            pltpu.SemaphoreType.DMA,
            pltpu.SemaphoreType.DMA,
        ],
    )
    def filter_kernel(src3, dst3, flags, csrc_o, cdst_o, cnt_o,
                      sin, din, posb, flb, dfill, cvec, shb, ssem, fsem):
        c = lax.axis_index("c")
        s = lax.axis_index("s")
        wid = s * NC + c

        pltpu.sync_copy(src3.at[pl.ds(wid * cpt, cpt)], sin)
        pltpu.sync_copy(dst3.at[pl.ds(wid * cpt, cpt)], din)

        dump_v = jnp.full((LN,), dump, I32)

        def pf(i, carry):
            dfill[pl.ds(i * LN, LN)] = dump_v
            return carry
        lax.fori_loop(0, cap // LN, pf, 0)
        rbase = wid * rcap
        pltpu.sync_copy(dfill, csrc_o.at[pl.ds(rbase, cap)])
        pltpu.sync_copy(dfill, cdst_o.at[pl.ds(rbase, cap)])

        def start_fgather(j, p):
            pltpu.make_async_copy(flags.at[din.at[j, 0]], flb.at[p, 0],
                                  fsem).start()

        def wait_fgather(p):
            pltpu.make_async_copy(flags.at[pl.ds(0, CH)], flb.at[p, 0],
                                  fsem).wait()

        start_fgather(0, 0)

        trash = rbase + cap
        lanes_zero = jnp.zeros((LN,), I32)
        shb[pl.ds(0, LN)] = lanes_zero

        def process(j, p, off):
            wait_fgather(p)

            @pl.when(j + 1 < cpt)
            def _():
                start_fgather(j + 1, 1 - p)
            for k in range(CH // LN):
                sl = pl.ds(k * LN, LN)
                fl = flb[p, 0, sl]
                m = fl > 0.0
                x = jnp.where(m, lanes_zero + 1, lanes_zero)
                for dshift in (1, 2, 4, 8):
                    shb[pl.ds(LN, LN)] = x
                    x = x + shb[pl.ds(LN - dshift, LN)]
                pos = jnp.where(m, rbase + off + x - 1, trash + lanes_zero)
                posb[j, 0, sl] = pos
                off = off + x[LN - 1]
            return off

        def body(jj, off):
            for p in range(2):
                off = process(jj * 2 + p, p, off)
            return off
        off = lax.fori_loop(0, cpt // 2, body, jnp.int32(0))
        if cpt % 2:
            off = process(cpt - 1, (cpt - 1) % 2, off)

        nch = (off + CH - 1) // CH
        cvec[...] = jnp.broadcast_to(nch, (LN,)).astype(I32)

        def sc(j, carry):
            pltpu.make_async_copy(sin.at[j, 0], csrc_o.at[posb.at[j, 0]],
                                  ssem).start()
            pltpu.make_async_copy(din.at[j, 0], cdst_o.at[posb.at[j, 0]],
                                  ssem).start()
            return carry
        lax.fori_loop(0, cpt, sc, 0)

        def dr(j, carry):
            pltpu.make_async_copy(sin.at[0, 0], csrc_o.at[pl.ds(0, CH)],
                                  ssem).wait()
            pltpu.make_async_copy(din.at[0, 0], cdst_o.at[pl.ds(0, CH)],
                                  ssem).wait()
            return carry
        lax.fori_loop(0, cpt, dr, 0)

        pltpu.sync_copy(cvec, cnt_o.at[pl.ds(wid * LN, LN)])

    return filter_kernel


def _make_agg_dyn_kernel(npad, n_chunk_rows):
    cpt = n_chunk_rows // NW
    zrows = npad // NS
    I32 = jnp.int32

    @functools.partial(
        pl.kernel,
        out_type=jax.ShapeDtypeStruct((NC * npad, 128), F32),
        mesh=_sc_mesh(),
        scratch_types=[
            pltpu.VMEM((1, 1, CH), I32),
            pltpu.VMEM((1, 1, CH), I32),
            pltpu.VMEM((CH, 128), F32),
            pltpu.VMEM((NW * LN,), I32),
            pltpu.VMEM_SHARED((npad, 128), F32),
            pltpu.SemaphoreType.DMA,
        ],
    )
    def agg_dyn_kernel(table, csrc3, cdst3, cnt16, out, sidx, didx, rows,
                       cnts_v, acc_sh, gsem):
        c = lax.axis_index("c")
        s = lax.axis_index("s")

        def zb(i, carry):
            for k in range(128 // LN):
                rows[i, pl.ds(k * LN, LN)] = jnp.zeros((LN,), F32)
            return carry
        lax.fori_loop(0, CH, zb, 0)
        for r in range(zrows // CH):
            pltpu.sync_copy(rows, acc_sh.at[pl.ds(s * zrows + r * CH, CH)])
        plsc.subcore_barrier()

        pltpu.sync_copy(cnt16, cnts_v)
        off = c * npad

        for q in range(2):
            r = 2 * s + q
            nch = cnts_v[pl.ds(r * LN, LN)][0]
            rbase = r * cpt

            def body(j, carry):
                pltpu.sync_copy(csrc3.at[pl.ds(rbase + j, 1)], sidx)
                pltpu.sync_copy(cdst3.at[pl.ds(rbase + j, 1)], didx)
                for k in range(CH // LN):
                    sl = pl.ds(k * LN, LN)
                    sidx[0, 0, sl] = sidx[0, 0, sl] + off
                pltpu.async_copy(table.at[sidx.at[0, 0]], rows, gsem).wait()
                pltpu.sync_copy(rows, acc_sh.at[didx.at[0, 0]], add=True)
                return carry
            lax.fori_loop(0, nch, body, 0)

        plsc.subcore_barrier()
        for r in range(zrows // 128):
            pltpu.sync_copy(acc_sh.at[pl.ds(s * zrows + r * 128, 128)],
                            out.at[pl.ds(c * npad + s * zrows + r * 128, 128)])

    return agg_dyn_kernel


def _make_tgather_kernel(npad, b):
    bpw = b // NW

    @functools.partial(
        pl.kernel,
        out_type=(
            jax.ShapeDtypeStruct((2, b, 128), F32),
            jax.ShapeDtypeStruct((2, b, 128), F32),
            jax.ShapeDtypeStruct((b,), F32),
        ),
        mesh=_sc_mesh(),
        scratch_types=[
            pltpu.VMEM((bpw,), jnp.int32),
            pltpu.VMEM((bpw,), jnp.int32),
            pltpu.VMEM((bpw, 128), F32),
            pltpu.VMEM((bpw, 128), F32),
            pltpu.VMEM((bpw, 128), F32),
            pltpu.VMEM((bpw, 128), F32),
            pltpu.VMEM((bpw,), F32),
            pltpu.SemaphoreType.DMA,
        ],
    )
    def tg_kernel(acc_t, hs_t, dinv_t, tgt, gacc, ghs, gdinv,
                  tidx, tidx_hi, ra, rb, rc, rd, dv, sem):
        c = lax.axis_index("c")
        s = lax.axis_index("s")
        wid = s * NC + c
        base = wid * bpw

        pltpu.sync_copy(tgt.at[pl.ds(base, bpw)], tidx)
        for k in range(bpw // LN):
            sl = pl.ds(k * LN, LN)
            tidx_hi[sl] = tidx[sl] + npad

        pltpu.async_copy(acc_t.at[tidx], ra, sem).wait()
        pltpu.async_copy(acc_t.at[tidx_hi], rb, sem).wait()
        pltpu.async_copy(hs_t.at[tidx], rc, sem).wait()
        pltpu.async_copy(hs_t.at[tidx_hi], rd, sem).wait()
        pltpu.async_copy(dinv_t.at[tidx], dv, sem).wait()

        pltpu.sync_copy(ra, gacc.at[0, pl.ds(base, bpw)])
        pltpu.sync_copy(rb, gacc.at[1, pl.ds(base, bpw)])
        pltpu.sync_copy(rc, ghs.at[0, pl.ds(base, bpw)])
        pltpu.sync_copy(rd, ghs.at[1, pl.ds(base, bpw)])
        pltpu.sync_copy(dv, gdinv.at[pl.ds(base, bpw)])

    return tg_kernel


def _mm_scale_body(x_ref, w_ref, ca_ref, cb_ref, hs_ref, dinv_ref):
    deg = ca_ref[...] + cb_ref[...] + 1.0
    dv = lax.rsqrt(deg)
    h = jnp.dot(x_ref[...], w_ref[...], preferred_element_type=F32)
    hs_ref[0] = dv * h
    dinv_ref[...] = dv


def _make_mm_scale(npad, d, blk):
    nb = npad // blk
    return pl.pallas_call(
        _mm_scale_body,
        grid=(nb, 2),
        in_specs=[
            pl.BlockSpec((blk, d), lambda i, c: (i, 0)),
            pl.BlockSpec((d, 128), lambda i, c: (0, c)),
            pl.BlockSpec((blk, 1), lambda i, c: (i, 0)),
            pl.BlockSpec((blk, 1), lambda i, c: (i, 0)),
        ],
        out_specs=[
            pl.BlockSpec((1, blk, 128), lambda i, c: (c, i, 0)),
            pl.BlockSpec((blk, 1), lambda i, c: (i, 0)),
        ],
        out_shape=[
            jax.ShapeDtypeStruct((2, npad, 128), F32),
            jax.ShapeDtypeStruct((npad, 1), F32),
        ],
    )


def _ew_body(acc_ref, hs_ref, dinv_ref, b_ref, out_ref):
    dv = dinv_ref[...]
    a = acc_ref[...] + hs_ref[...]
    h1 = jnp.maximum(dv * a + b_ref[0], 0.0)
    out_ref[...] = dv * h1


def _make_ew(npad, blk):
    nb = npad // blk
    return pl.pallas_call(
        _ew_body,
        grid=(2, nb),
        in_specs=[
            pl.BlockSpec((blk, 128), lambda c, i: (c * nb + i, 0)),
            pl.BlockSpec((blk, 128), lambda c, i: (c * nb + i, 0)),
            pl.BlockSpec((blk, 1), lambda c, i: (i, 0)),
            pl.BlockSpec((1, 1, 128), lambda c, i: (c, 0, 0)),
        ],
        out_specs=pl.BlockSpec((blk, 128), lambda c, i: (c * nb + i, 0)),
        out_shape=jax.ShapeDtypeStruct((2 * npad, 128), F32),
    )


def _head_body(gacc_ref, ghs_ref, gdinv_ref, w2t_ref, b2_ref, wih_ref,
               bih_ref, bhh_ref, fcw_ref, fcb_ref, out_ref):
    ga = gacc_ref[...]
    gh = ghs_ref[...]
    gsum = jnp.concatenate([ga[0] + gh[0], ga[1] + gh[1]], axis=1)
    tpre = gdinv_ref[...] * gsum
    t = jnp.maximum(jnp.dot(tpre, w2t_ref[...], preferred_element_type=F32)
                    + b2_ref[...], 0.0)
    gi = jnp.dot(t, wih_ref[...], preferred_element_type=F32) + bih_ref[...]
    bhh = bhh_ref[...]
    gh_dim = t.shape[1]
    i_r = gi[:, :gh_dim]
    i_z = gi[:, gh_dim:2 * gh_dim]
    i_n = gi[:, 2 * gh_dim:]
    h_r = bhh[:, :gh_dim]
    h_z = bhh[:, gh_dim:2 * gh_dim]
    h_n = bhh[:, 2 * gh_dim:]
    r = jax.nn.sigmoid(i_r + h_r)
    z = jax.nn.sigmoid(i_z + h_z)
    n_ = jnp.tanh(i_n + r * h_n)
    hN = (1.0 - z) * n_
    out_ref[...] = jnp.dot(hN, fcw_ref[...], preferred_element_type=F32) + fcb_ref[...]


def _make_head(b, h):
    return pl.pallas_call(
        _head_body,
        out_shape=jax.ShapeDtypeStruct((b, 128), F32),
    )


def kernel(x, edge_index, target_node_index, W1, b1, W2, b2,
           W_ih, W_hh, b_ih, b_hh, fc_W, fc_b):
    n, d = x.shape
    e = edge_index.shape[1]
    b = target_node_index.shape[0]
    h = W1.shape[0]
    c_out = fc_W.shape[0]

    npad = ((n + 1 + 4095) // 4096) * 4096
    dump = n
    epad = ((e + NW * CH - 1) // (NW * CH)) * (NW * CH)
    n_chunk_rows = epad // CH

    i32 = jnp.int32
    src = edge_index[0]
    dst = edge_index[1]
    padlen = epad - e
    src3 = jnp.concatenate(
        [src, jnp.full((padlen,), dump, i32)]).reshape(n_chunk_rows, 1, CH)
    dst3 = jnp.concatenate(
        [dst, jnp.full((padlen,), dump, i32)]).reshape(n_chunk_rows, 1, CH)

    x_pad = jnp.pad(x, ((0, npad - n), (0, 0)))
    w1t = W1.T
    w2t = W2.T
    wih_t = W_ih.T
    fcw_t = jnp.pad(fc_W.T, ((0, 0), (0, 128 - c_out)))
    fcb_p = jnp.pad(fc_b, (0, 128 - c_out)).reshape(1, 128)
    b1r = b1.reshape(2, 1, 128)
    b2r = b2.reshape(1, h)
    bihr = b_ih.reshape(1, 3 * h)
    bhhr = b_hh.reshape(1, 3 * h)

    cnt, tflags = _make_deg_kernel(npad, n_chunk_rows, b)(
        dst3, target_node_index)
    ca = cnt[:npad].reshape(npad, 1)
    cb = cnt[npad:].reshape(npad, 1)

    hs3, dinv = _make_mm_scale(npad, d, 512)(x_pad, w1t, ca, cb)
    hs = hs3.reshape(2 * npad, 128)

    agg = _make_agg_kernel(npad, n_chunk_rows)
    acc1 = agg(hs, src3, dst3)

    hs2 = _make_ew(npad, 512)(acc1, hs, dinv, b1r)

    csrc_f, cdst_f, cnts = _make_filter_kernel(npad, n_chunk_rows, b, dump)(
        src3, dst3, tflags)
    cpt = n_chunk_rows // NW
    rcap = (cpt + 1) * CH
    csrc3 = csrc_f.reshape(NW, rcap)[:, :cpt * CH].reshape(n_chunk_rows, 1, CH)
    cdst3 = cdst_f.reshape(NW, rcap)[:, :cpt * CH].reshape(n_chunk_rows, 1, CH)
    acc2 = _make_agg_dyn_kernel(npad, n_chunk_rows)(hs2, csrc3, cdst3, cnts)

    gacc, ghs, gdinv = _make_tgather_kernel(npad, b)(
        acc2, hs2, dinv.reshape(npad), target_node_index)

    out128 = _make_head(b, h)(gacc, ghs, gdinv.reshape(b, 1), w2t, b2r,
                              wih_t, bihr, bhhr, fcw_t, fcb_p)
    return out128[:, :c_out]


# speedup vs baseline: 4.7454x; 4.7454x over previous
import functools
import jax
import jax.numpy as jnp
from jax import lax
from jax.experimental import pallas as pl
from jax.experimental.pallas import tpu as pltpu
from jax.experimental.pallas import tpu_sc as plsc

NC = 2
NS = 16
NW = NC * NS
LN = 16

F32 = jnp.float32


def _sc_mesh():
    return plsc.VectorSubcoreMesh(core_axis_name="c", subcore_axis_name="s",
                                  num_cores=NC, num_subcores=NS)


CH = 96


def _make_deg_kernel(npad, n_chunk_rows, b):
    rows_per_tile = n_chunk_rows // NW
    zrows = npad // NS
    bps = b // NS

    @functools.partial(
        pl.kernel,
        out_type=(
            jax.ShapeDtypeStruct((NC * npad,), F32),
            jax.ShapeDtypeStruct((npad,), F32),
        ),
        mesh=_sc_mesh(),
        scratch_types=[
            pltpu.VMEM((rows_per_tile, 1, CH), jnp.int32),
            pltpu.VMEM((CH,), F32),
            pltpu.VMEM((zrows,), F32),
            pltpu.VMEM((bps,), jnp.int32),
            pltpu.VMEM_SHARED((npad,), F32),
            pltpu.VMEM_SHARED((npad,), F32),
        ],
    )
    def deg_kernel(dst3, tgt, out, flags_o, idx_v, ones_v, zbuf, tgt_v,
                   cnt_sh, flag_sh):
        c = lax.axis_index("c")
        s = lax.axis_index("s")
        wid = s * NC + c

        def zb(i, carry):
            zbuf[pl.ds(i * LN, LN)] = jnp.zeros((LN,), F32)
            return carry
        lax.fori_loop(0, zrows // LN, zb, 0)
        for k in range(CH // LN):
            ones_v[pl.ds(k * LN, LN)] = jnp.ones((LN,), F32)
        pltpu.sync_copy(zbuf, cnt_sh.at[pl.ds(s * zrows, zrows)])
        pltpu.sync_copy(zbuf, flag_sh.at[pl.ds(s * zrows, zrows)])
        plsc.subcore_barrier()

        pltpu.sync_copy(tgt.at[pl.ds(s * bps, bps)], tgt_v)
        pltpu.sync_copy(ones_v.at[pl.ds(0, bps)], flag_sh.at[tgt_v], add=True)

        pltpu.sync_copy(dst3.at[pl.ds(wid * rows_per_tile, rows_per_tile)], idx_v)

        def body(j, carry):
            pltpu.sync_copy(ones_v, cnt_sh.at[idx_v.at[j, 0]], add=True)
            return carry
        lax.fori_loop(0, rows_per_tile, body, 0)

        plsc.subcore_barrier()
        pltpu.sync_copy(cnt_sh.at[pl.ds(s * zrows, zrows)],
                        out.at[pl.ds(c * npad + s * zrows, zrows)])

        @pl.when(c == 0)
        def _():
            pltpu.sync_copy(flag_sh.at[pl.ds(s * zrows, zrows)],
                            flags_o.at[pl.ds(s * zrows, zrows)])

    return deg_kernel


def _make_agg_kernel(npad, n_chunk_rows):
    rows_per_tile = n_chunk_rows // NS
    zrows = npad // NS

    @functools.partial(
        pl.kernel,
        out_type=jax.ShapeDtypeStruct((NC * npad, 128), F32),
        mesh=_sc_mesh(),
        scratch_types=[
            pltpu.VMEM((2, 1, CH), jnp.int32),
            pltpu.VMEM((2, 1, CH), jnp.int32),
            pltpu.VMEM((2, CH, 128), F32),
            pltpu.VMEM_SHARED((npad, 128), F32),
            pltpu.SemaphoreType.DMA,
            pltpu.SemaphoreType.DMA,
        ],
    )
    def agg_kernel(table, src3, dst3, out, sidx, didx, rows, acc_sh,
                   gsem, isem):
        c = lax.axis_index("c")
        s = lax.axis_index("s")

        def zb(i, carry):
            for k in range(128 // LN):
                rows[0, i, pl.ds(k * LN, LN)] = jnp.zeros((LN,), F32)
            return carry
        lax.fori_loop(0, CH, zb, 0)
        for r in range(zrows // CH):
            pltpu.sync_copy(rows.at[0], acc_sh.at[pl.ds(s * zrows + r * CH, CH)])
        plsc.subcore_barrier()

        base = s * rows_per_tile
        off = c * npad

        def fetch(j, p):
            pltpu.make_async_copy(src3.at[pl.ds(base + j, 1)],
                                  sidx.at[pl.ds(p, 1)], isem).start()
            pltpu.make_async_copy(dst3.at[pl.ds(base + j, 1)],
                                  didx.at[pl.ds(p, 1)], isem).start()

        def drain_idx():
            pltpu.make_async_copy(src3.at[pl.ds(0, 1)],
                                  sidx.at[pl.ds(0, 1)], isem).wait()
            pltpu.make_async_copy(dst3.at[pl.ds(0, 1)],
                                  didx.at[pl.ds(0, 1)], isem).wait()

        def shift(p):
            for k in range(CH // LN):
                sl = pl.ds(k * LN, LN)
                sidx[p, 0, sl] = sidx[p, 0, sl] + off

        def start_gather(p):
            pltpu.make_async_copy(table.at[sidx.at[p, 0]], rows.at[p],
                                  gsem).start()

        def wait_gather(p):
            pltpu.make_async_copy(table.at[pl.ds(0, CH)], rows.at[p],
                                  gsem).wait()

        fetch(0, 0)
        drain_idx()
        shift(0)
        start_gather(0)
        fetch(1, 1)

        def body(jj, carry):
            for p in range(2):
                j = jj * 2 + p
                wait_gather(p)

                @pl.when(j + 1 < rows_per_tile)
                def _():
                    drain_idx()
                    shift(1 - p)
                    start_gather(1 - p)
                pltpu.sync_copy(rows.at[p], acc_sh.at[didx.at[p, 0]], add=True)

                @pl.when(j + 2 < rows_per_tile)
                def _():
                    fetch(j + 2, p)
            return carry
        lax.fori_loop(0, rows_per_tile // 2, body, 0)

        plsc.subcore_barrier()
        for r in range(zrows // 128):
            pltpu.sync_copy(acc_sh.at[pl.ds(s * zrows + r * 128, 128)],
                            out.at[pl.ds(c * npad + s * zrows + r * 128, 128)])

    return agg_kernel


def _make_filter_kernel(npad, n_chunk_rows, b, dump):
    cpt = n_chunk_rows // NW
    cap = cpt * CH
    I32 = jnp.int32

    rcap = cap + CH

    @functools.partial(
        pl.kernel,
        out_type=(
            jax.ShapeDtypeStruct((NW * rcap,), I32),
            jax.ShapeDtypeStruct((NW * rcap,), I32),
            jax.ShapeDtypeStruct((NW * LN,), I32),
        ),
        mesh=_sc_mesh(),
        scratch_types=[
            pltpu.VMEM((cpt, 1, CH), I32),
            pltpu.VMEM((cpt, 1, CH), I32),
            pltpu.VMEM((cpt, 1, CH), I32),
            pltpu.VMEM((2, 1, CH), F32),
            pltpu.VMEM((cap,), I32),
            pltpu.VMEM((LN,), I32),
            pltpu.VMEM((3 * LN,), I32),
            pltpu.VMEM_SHARED((NS * rcap,), I32),
            pltpu.VMEM_SHARED((NS * rcap,), I32),
            pltpu.SemaphoreType.DMA,
            pltpu.SemaphoreType.DMA,
        ],
    )
    def filter_kernel(src3, dst3, flags, csrc_o, cdst_o, cnt_o,
                      sin, din, posb, flb, dfill, cvec, shb, csr_sh, cds_sh,
                      ssem, fsem):
        c = lax.axis_index("c")
        s = lax.axis_index("s")
        wid = s * NC + c

        pltpu.sync_copy(src3.at[pl.ds(wid * cpt, cpt)], sin)
        pltpu.sync_copy(dst3.at[pl.ds(wid * cpt, cpt)], din)

        dump_v = jnp.full((LN,), dump, I32)

        def pf(i, carry):
            dfill[pl.ds(i * LN, LN)] = dump_v
            return carry
        lax.fori_loop(0, cap // LN, pf, 0)
        lbase = s * rcap
        pltpu.sync_copy(dfill, csr_sh.at[pl.ds(lbase, cap)])
        pltpu.sync_copy(dfill, cds_sh.at[pl.ds(lbase, cap)])

        def start_fgather(j, p):
            pltpu.make_async_copy(flags.at[din.at[j, 0]], flb.at[p, 0],
                                  fsem).start()

        def wait_fgather(p):
            pltpu.make_async_copy(flags.at[pl.ds(0, CH)], flb.at[p, 0],
                                  fsem).wait()

        start_fgather(0, 0)

        trash = lbase + cap
        lanes_zero = jnp.zeros((LN,), I32)
        shb[pl.ds(0, LN)] = lanes_zero
        shb[pl.ds(2 * LN, LN)] = lanes_zero

        def process2(j, p, off_v):
            wait_fgather(p)

            @pl.when(j + 1 < cpt)
            def _():
                start_fgather(j + 1, 1 - p)
            for k in range(CH // LN):
                sl = pl.ds(k * LN, LN)
                fl = flb[p, 0, sl]
                m = fl > 0.0
                x0 = jnp.where(m, lanes_zero + 1, lanes_zero)
                x = x0
                for dshift in (1, 2, 4, 8):
                    shb[pl.ds(LN, LN)] = x
                    x = x + shb[pl.ds(LN - dshift, LN)]
                t = x0
                for dshift in (1, 2, 4, 8):
                    shb[pl.ds(LN, LN)] = t
                    t = t + shb[pl.ds(LN + dshift, LN)]
                total = x + t - x0
                pos = jnp.where(m, lbase + off_v + x - 1, trash + lanes_zero)
                posb[j, 0, sl] = pos
                off_v = off_v + total
            return off_v

        def body(jj, off_v):
            for p in range(2):
                off_v = process2(jj * 2 + p, p, off_v)
            return off_v
        off_v = lax.fori_loop(0, cpt // 2, body, jnp.zeros((LN,), I32))
        if cpt % 2:
            off_v = process2(cpt - 1, (cpt - 1) % 2, off_v)

        nch = (off_v[0] + CH - 1) // CH
        cvec[...] = jnp.broadcast_to(nch, (LN,)).astype(I32)

        def sc(j, carry):
            pltpu.make_async_copy(sin.at[j, 0], csr_sh.at[posb.at[j, 0]],
                                  ssem).start()
            pltpu.make_async_copy(din.at[j, 0], cds_sh.at[posb.at[j, 0]],
                                  ssem).start()
            return carry
        lax.fori_loop(0, cpt, sc, 0)

        def dr(j, carry):
            pltpu.make_async_copy(sin.at[0, 0], csr_sh.at[pl.ds(0, CH)],
                                  ssem).wait()
            pltpu.make_async_copy(din.at[0, 0], cds_sh.at[pl.ds(0, CH)],
                                  ssem).wait()
            return carry
        lax.fori_loop(0, cpt, dr, 0)


        pltpu.sync_copy(csr_sh.at[pl.ds(lbase, cap)], dfill)
        pltpu.sync_copy(dfill, csrc_o.at[pl.ds(wid * rcap, cap)])
        pltpu.sync_copy(cds_sh.at[pl.ds(lbase, cap)], dfill)
        pltpu.sync_copy(dfill, cdst_o.at[pl.ds(wid * rcap, cap)])
        pltpu.sync_copy(cvec, cnt_o.at[pl.ds(wid * LN, LN)])

    return filter_kernel


def _make_agg_dyn_kernel(npad, n_chunk_rows):
    cpt = n_chunk_rows // NW
    zrows = npad // NS
    I32 = jnp.int32

    @functools.partial(
        pl.kernel,
        out_type=jax.ShapeDtypeStruct((NC * npad, 128), F32),
        mesh=_sc_mesh(),
        scratch_types=[
            pltpu.VMEM((1, 1, CH), I32),
            pltpu.VMEM((1, 1, CH), I32),
            pltpu.VMEM((CH, 128), F32),
            pltpu.VMEM((NW * LN,), I32),
            pltpu.VMEM_SHARED((npad, 128), F32),
            pltpu.SemaphoreType.DMA,
        ],
    )
    def agg_dyn_kernel(table, csrc3, cdst3, cnt16, out, sidx, didx, rows,
                       cnts_v, acc_sh, gsem):
        c = lax.axis_index("c")
        s = lax.axis_index("s")

        def zb(i, carry):
            for k in range(128 // LN):
                rows[i, pl.ds(k * LN, LN)] = jnp.zeros((LN,), F32)
            return carry
        lax.fori_loop(0, CH, zb, 0)
        for r in range(zrows // CH):
            pltpu.sync_copy(rows, acc_sh.at[pl.ds(s * zrows + r * CH, CH)])
        plsc.subcore_barrier()

        pltpu.sync_copy(cnt16, cnts_v)
        off = c * npad

        for q in range(2):
            r = 2 * s + q
            nch = cnts_v[pl.ds(r * LN, LN)][0]
            rbase = r * cpt

            def body(j, carry):
                pltpu.sync_copy(csrc3.at[pl.ds(rbase + j, 1)], sidx)
                pltpu.sync_copy(cdst3.at[pl.ds(rbase + j, 1)], didx)
                for k in range(CH // LN):
                    sl = pl.ds(k * LN, LN)
                    sidx[0, 0, sl] = sidx[0, 0, sl] + off
                pltpu.async_copy(table.at[sidx.at[0, 0]], rows, gsem).wait()
                pltpu.sync_copy(rows, acc_sh.at[didx.at[0, 0]], add=True)
                return carry
            lax.fori_loop(0, nch, body, 0)

        plsc.subcore_barrier()
        for r in range(zrows // 128):
            pltpu.sync_copy(acc_sh.at[pl.ds(s * zrows + r * 128, 128)],
                            out.at[pl.ds(c * npad + s * zrows + r * 128, 128)])

    return agg_dyn_kernel


def _make_tgather_kernel(npad, b):
    bpw = b // NW

    @functools.partial(
        pl.kernel,
        out_type=(
            jax.ShapeDtypeStruct((2, b, 128), F32),
            jax.ShapeDtypeStruct((2, b, 128), F32),
            jax.ShapeDtypeStruct((b,), F32),
        ),
        mesh=_sc_mesh(),
        scratch_types=[
            pltpu.VMEM((bpw,), jnp.int32),
            pltpu.VMEM((bpw,), jnp.int32),
            pltpu.VMEM((bpw, 128), F32),
            pltpu.VMEM((bpw, 128), F32),
            pltpu.VMEM((bpw, 128), F32),
            pltpu.VMEM((bpw, 128), F32),
            pltpu.VMEM((bpw,), F32),
            pltpu.SemaphoreType.DMA,
        ],
    )
    def tg_kernel(acc_t, hs_t, dinv_t, tgt, gacc, ghs, gdinv,
                  tidx, tidx_hi, ra, rb, rc, rd, dv, sem):
        c = lax.axis_index("c")
        s = lax.axis_index("s")
        wid = s * NC + c
        base = wid * bpw

        pltpu.sync_copy(tgt.at[pl.ds(base, bpw)], tidx)
        for k in range(bpw // LN):
            sl = pl.ds(k * LN, LN)
            tidx_hi[sl] = tidx[sl] + npad

        pltpu.async_copy(acc_t.at[tidx], ra, sem).wait()
        pltpu.async_copy(acc_t.at[tidx_hi], rb, sem).wait()
        pltpu.async_copy(hs_t.at[tidx], rc, sem).wait()
        pltpu.async_copy(hs_t.at[tidx_hi], rd, sem).wait()
        pltpu.async_copy(dinv_t.at[tidx], dv, sem).wait()

        pltpu.sync_copy(ra, gacc.at[0, pl.ds(base, bpw)])
        pltpu.sync_copy(rb, gacc.at[1, pl.ds(base, bpw)])
        pltpu.sync_copy(rc, ghs.at[0, pl.ds(base, bpw)])
        pltpu.sync_copy(rd, ghs.at[1, pl.ds(base, bpw)])
        pltpu.sync_copy(dv, gdinv.at[pl.ds(base, bpw)])

    return tg_kernel


def _mm_scale_body(x_ref, w_ref, ca_ref, cb_ref, hs_ref, dinv_ref):
    deg = ca_ref[...] + cb_ref[...] + 1.0
    dv = lax.rsqrt(deg)
    h = jnp.dot(x_ref[...], w_ref[...], preferred_element_type=F32)
    hs_ref[0] = dv * h
    dinv_ref[...] = dv


def _make_mm_scale(npad, d, blk):
    nb = npad // blk
    return pl.pallas_call(
        _mm_scale_body,
        grid=(nb, 2),
        in_specs=[
            pl.BlockSpec((blk, d), lambda i, c: (i, 0)),
            pl.BlockSpec((d, 128), lambda i, c: (0, c)),
            pl.BlockSpec((blk, 1), lambda i, c: (i, 0)),
            pl.BlockSpec((blk, 1), lambda i, c: (i, 0)),
        ],
        out_specs=[
            pl.BlockSpec((1, blk, 128), lambda i, c: (c, i, 0)),
            pl.BlockSpec((blk, 1), lambda i, c: (i, 0)),
        ],
        out_shape=[
            jax.ShapeDtypeStruct((2, npad, 128), F32),
            jax.ShapeDtypeStruct((npad, 1), F32),
        ],
    )


def _ew_body(acc_ref, hs_ref, dinv_ref, b_ref, out_ref):
    dv = dinv_ref[...]
    a = acc_ref[...] + hs_ref[...]
    h1 = jnp.maximum(dv * a + b_ref[0], 0.0)
    out_ref[...] = dv * h1


def _make_ew(npad, blk):
    nb = npad // blk
    return pl.pallas_call(
        _ew_body,
        grid=(2, nb),
        in_specs=[
            pl.BlockSpec((blk, 128), lambda c, i: (c * nb + i, 0)),
            pl.BlockSpec((blk, 128), lambda c, i: (c * nb + i, 0)),
            pl.BlockSpec((blk, 1), lambda c, i: (i, 0)),
            pl.BlockSpec((1, 1, 128), lambda c, i: (c, 0, 0)),
        ],
        out_specs=pl.BlockSpec((blk, 128), lambda c, i: (c * nb + i, 0)),
        out_shape=jax.ShapeDtypeStruct((2 * npad, 128), F32),
    )


def _head_body(gacc_ref, ghs_ref, gdinv_ref, w2t_ref, b2_ref, wih_ref,
               bih_ref, bhh_ref, fcw_ref, fcb_ref, out_ref):
    ga = gacc_ref[...]
    gh = ghs_ref[...]
    gsum = jnp.concatenate([ga[0] + gh[0], ga[1] + gh[1]], axis=1)
    tpre = gdinv_ref[...] * gsum
    t = jnp.maximum(jnp.dot(tpre, w2t_ref[...], preferred_element_type=F32)
                    + b2_ref[...], 0.0)
    gi = jnp.dot(t, wih_ref[...], preferred_element_type=F32) + bih_ref[...]
    bhh = bhh_ref[...]
    gh_dim = t.shape[1]
    i_r = gi[:, :gh_dim]
    i_z = gi[:, gh_dim:2 * gh_dim]
    i_n = gi[:, 2 * gh_dim:]
    h_r = bhh[:, :gh_dim]
    h_z = bhh[:, gh_dim:2 * gh_dim]
    h_n = bhh[:, 2 * gh_dim:]
    r = jax.nn.sigmoid(i_r + h_r)
    z = jax.nn.sigmoid(i_z + h_z)
    n_ = jnp.tanh(i_n + r * h_n)
    hN = (1.0 - z) * n_
    out_ref[...] = jnp.dot(hN, fcw_ref[...], preferred_element_type=F32) + fcb_ref[...]


def _make_head(b, h):
    return pl.pallas_call(
        _head_body,
        out_shape=jax.ShapeDtypeStruct((b, 128), F32),
    )


def kernel(x, edge_index, target_node_index, W1, b1, W2, b2,
           W_ih, W_hh, b_ih, b_hh, fc_W, fc_b):
    n, d = x.shape
    e = edge_index.shape[1]
    b = target_node_index.shape[0]
    h = W1.shape[0]
    c_out = fc_W.shape[0]

    npad = ((n + 1 + 4095) // 4096) * 4096
    dump = n
    epad = ((e + NW * CH - 1) // (NW * CH)) * (NW * CH)
    n_chunk_rows = epad // CH

    i32 = jnp.int32
    src = edge_index[0]
    dst = edge_index[1]
    padlen = epad - e
    src3 = jnp.concatenate(
        [src, jnp.full((padlen,), dump, i32)]).reshape(n_chunk_rows, 1, CH)
    dst3 = jnp.concatenate(
        [dst, jnp.full((padlen,), dump, i32)]).reshape(n_chunk_rows, 1, CH)

    x_pad = jnp.pad(x, ((0, npad - n), (0, 0)))
    w1t = W1.T
    w2t = W2.T
    wih_t = W_ih.T
    fcw_t = jnp.pad(fc_W.T, ((0, 0), (0, 128 - c_out)))
    fcb_p = jnp.pad(fc_b, (0, 128 - c_out)).reshape(1, 128)
    b1r = b1.reshape(2, 1, 128)
    b2r = b2.reshape(1, h)
    bihr = b_ih.reshape(1, 3 * h)
    bhhr = b_hh.reshape(1, 3 * h)

    cnt, tflags = _make_deg_kernel(npad, n_chunk_rows, b)(
        dst3, target_node_index)
    ca = cnt[:npad].reshape(npad, 1)
    cb = cnt[npad:].reshape(npad, 1)

    hs3, dinv = _make_mm_scale(npad, d, 512)(x_pad, w1t, ca, cb)
    hs = hs3.reshape(2 * npad, 128)

    agg = _make_agg_kernel(npad, n_chunk_rows)
    acc1 = agg(hs, src3, dst3)

    hs2 = _make_ew(npad, 512)(acc1, hs, dinv, b1r)

    csrc_f, cdst_f, cnts = _make_filter_kernel(npad, n_chunk_rows, b, dump)(
        src3, dst3, tflags)
    cpt = n_chunk_rows // NW
    rcap = (cpt + 1) * CH
    csrc3 = csrc_f.reshape(NW, rcap)[:, :cpt * CH].reshape(n_chunk_rows, 1, CH)
    cdst3 = cdst_f.reshape(NW, rcap)[:, :cpt * CH].reshape(n_chunk_rows, 1, CH)
    acc2 = _make_agg_dyn_kernel(npad, n_chunk_rows)(hs2, csrc3, cdst3, cnts)

    gacc, ghs, gdinv = _make_tgather_kernel(npad, b)(
        acc2, hs2, dinv.reshape(npad), target_node_index)

    out128 = _make_head(b, h)(gacc, ghs, gdinv.reshape(b, 1), w2t, b2r,
                              wih_t, bihr, bhhr, fcw_t, fcb_p)
    return out128[:, :c_out]
